# baseline (speedup 1.0000x reference)
import functools

import jax
import jax.numpy as jnp
from jax import lax
from jax.experimental import pallas as pl
from jax.experimental.pallas import tpu as pltpu
from jax.experimental.pallas import tpu_sc as plsc

E = 16
C = 256
K = 512
S = 512
B = 8
T = 262144
VOCAB = 257
NTOK = B * T
NPAIR = NTOK // 2
NWIN = B * (T // S)
KE = K * E

NC = 2
NS = 16
NW = NC * NS
PER_W = NTOK // NW
PAIR_W = PER_W // 2
CHUNK = 1024
NCHUNK = PAIR_W // CHUNK
IDX_GRP = PAIR_W // 16

BM = 512
GRID = NWIN // BM


def _sc_gather_body(x_hbm, table_hbm, out_hbm, xbuf, rows0, rows1,
                    isem, gsem0, gsem1, osem0, osem1):
    wid = lax.axis_index("s") * NC + lax.axis_index("c")
    rows = (rows0, rows1)
    gsem = (gsem0, gsem1)
    osem = (osem0, osem1)

    pltpu.async_copy(x_hbm.at[pl.ds(wid * PAIR_W, PAIR_W)], xbuf, isem).wait()

    pbase = wid * PAIR_W

    def gather_start(c, b):
        return pltpu.async_copy(
            table_hbm.at[xbuf.at[pl.ds(c * CHUNK, CHUNK)]], rows[b], gsem[b])

    def out_start(c, b):
        return pltpu.async_copy(
            rows[b], out_hbm.at[pl.ds(pbase + c * CHUNK, CHUNK)], osem[b])

    def out_wait(c, b):
        pltpu.make_async_copy(
            rows[b], out_hbm.at[pl.ds(pbase + c * CHUNK, CHUNK)],
            osem[b]).wait()

    g0 = gather_start(0, 0)
    g1 = gather_start(1, 1)
    g0.wait()
    out_start(0, 0)
    g1.wait()
    out_start(1, 1)

    def lap(g, carry):
        for b in range(2):
            c = 2 * g + b
            out_wait(c - 2, b)
            gather_start(c, b).wait()
            out_start(c, b)
        return carry

    lax.fori_loop(1, NCHUNK // 2, lap, 0)
    for b in range(2):
        out_wait(NCHUNK - 2 + b, b)


@functools.cache
def _sc_gather():
    return pl.kernel(
        _sc_gather_body,
        out_type=jax.ShapeDtypeStruct((NPAIR, E), jnp.int32),
        mesh=plsc.VectorSubcoreMesh(core_axis_name="c", subcore_axis_name="s"),
        scratch_types=[
            pltpu.VMEM((PAIR_W,), jnp.int32),
            pltpu.VMEM((CHUNK, E), jnp.int32),
            pltpu.VMEM((CHUNK, E), jnp.int32),
            pltpu.SemaphoreType.DMA,
            pltpu.SemaphoreType.DMA,
            pltpu.SemaphoreType.DMA,
            pltpu.SemaphoreType.DMA,
            pltpu.SemaphoreType.DMA,
        ],
        compiler_params=pltpu.CompilerParams(use_tc_tiling_on_sc=False,
                                             needs_layout_passes=False),
    )


def _tc_body(z_ref, w_ref, b_ref, ws_ref, bs_ref, gw_ref, gb_ref,
             f1w_ref, f1b_ref, f2w_ref, f2b_ref, out_ref, m1_ref, m2_ref):
    i = pl.program_id(0)

    @pl.when(i == 0)
    def _init():
        m1_ref[...] = jnp.full((B, C), -jnp.inf, jnp.float32)
        m2_ref[...] = jnp.full((B, C), -jnp.inf, jnp.float32)

    a = z_ref[...]
    c = jnp.dot(a, w_ref[...], preferred_element_type=jnp.float32)
    c = c + b_ref[...]
    u = c[:, :C] * jax.nn.sigmoid(c[:, C:2 * C])
    s = jnp.dot(u, ws_ref[...], preferred_element_type=jnp.float32)
    s = s + bs_ref[...]
    s = jnp.where(s >= 0.0, s, 0.01 * s)
    v = c[:, 2 * C:3 * C] * jax.nn.sigmoid(c[:, 3 * C:])

    m1_blk = jnp.max(s, axis=0, keepdims=True)
    m2_blk = jnp.max(v, axis=0, keepdims=True)
    row = lax.broadcasted_iota(jnp.int32, (B, 1), 0)
    sel = row == i
    m1_ref[...] = jnp.where(sel, jnp.maximum(m1_ref[...], m1_blk), m1_ref[...])
    m2_ref[...] = jnp.where(sel, jnp.maximum(m2_ref[...], m2_blk), m2_ref[...])

    @pl.when(i == pl.num_programs(0) - 1)
    def _head():
        gates = jax.nn.sigmoid(
            jnp.dot(m1_ref[...], gw_ref[...],
                    preferred_element_type=jnp.float32) + gb_ref[...])
        pooled = m2_ref[...] * gates
        f = jnp.dot(pooled, f1w_ref[...], preferred_element_type=jnp.float32)
        f = jnp.maximum(f + f1b_ref[...], 0.0)
        o = jnp.dot(f, f2w_ref[...], preferred_element_type=jnp.float32)
        out_ref[...] = o + f2b_ref[...]


def _full(shape):
    return pl.BlockSpec(shape, lambda i: (0, 0))


_tc_call = pl.pallas_call(
    _tc_body,
    grid=(GRID,),
    in_specs=[
        pl.BlockSpec((BM, KE), lambda i: (i, 0)),
        _full((KE, 4 * C)),
        _full((1, 4 * C)),
        _full((C, C)),
        _full((1, C)),
        _full((C, C)),
        _full((1, C)),
        _full((C, C)),
        _full((1, C)),
        _full((C, 128)),
        _full((1, 128)),
    ],
    out_specs=pl.BlockSpec((B, 128), lambda i: (0, 0)),
    out_shape=jax.ShapeDtypeStruct((B, 128), jnp.float32),
    scratch_shapes=[
        pltpu.VMEM((B, C), jnp.float32),
        pltpu.VMEM((B, C), jnp.float32),
    ],
)


def kernel(x, embed, ctx_conv_w, ctx_conv_b, ctx_share_w, ctx_share_b,
           gcg_conv_w, gcg_conv_b, gate_w, gate_b,
           fc1_w, fc1_b, fc2_w, fc2_b):
    emb16 = embed.astype(jnp.bfloat16)
    left = jnp.broadcast_to(emb16[:, None, :], (VOCAB, VOCAB, E))
    right = jnp.broadcast_to(emb16[None, :, :], (VOCAB, VOCAB, E))
    pair = jnp.concatenate([left, right], axis=-1).reshape(VOCAB * VOCAB, E, 2)
    pair_i32 = lax.bitcast_convert_type(pair, jnp.int32)

    xp = x.reshape(NPAIR, 2)
    ix = xp[:, 0] * VOCAB + xp[:, 1]

    zp = _sc_gather()(ix, pair_i32)
    z = lax.bitcast_convert_type(zp, jnp.bfloat16).reshape(NWIN, KE)

    wc = ctx_conv_w.transpose(2, 1, 0).reshape(KE, 2 * C)
    wg = gcg_conv_w.transpose(2, 1, 0).reshape(KE, 2 * C)
    w_all = jnp.concatenate([wc, wg], axis=1).astype(jnp.bfloat16)
    b_all = jnp.concatenate([ctx_conv_b, gcg_conv_b])[None, :]
    ws = ctx_share_w[:, :, 0].T
    bs = ctx_share_b[None, :]
    gw = gate_w.T
    gb = gate_b[None, :]
    f1w = fc1_w.T
    f1b = fc1_b[None, :]
    f2w = jnp.pad(fc2_w.T, ((0, 0), (0, 128 - fc2_w.shape[0])))
    f2b = jnp.pad(fc2_b, (0, 128 - fc2_b.shape[0]))[None, :]

    out = _tc_call(z, w_all, b_all, ws, bs, gw, gb, f1w, f1b, f2w, f2b)
    return out[:, :fc2_w.shape[0]]


# speedup vs baseline: 25.0205x; 25.0205x over previous
import functools

import jax
import jax.numpy as jnp
from jax import lax
from jax.experimental import pallas as pl
from jax.experimental.pallas import tpu as pltpu
from jax.experimental.pallas import tpu_sc as plsc

E = 16
C = 256
K = 512
S = 512
B = 8
T = 262144
VOCAB = 257
NTOK = B * T
NWIN = B * (T // S)
KE = K * E

NC = 2
NS = 16
NW = NC * NS
PER_W = NTOK // NW
CHUNK = 1024
NCHUNK = PER_W // CHUNK

BM = 512
GRID = NWIN // BM


def _sc_gather_body(x_hbm, table_hbm, out_hbm, xbuf, tbl, rows0, rows1,
                    isem, tsem, osem0, osem1):
    wid = lax.axis_index("s") * NC + lax.axis_index("c")
    base0 = wid * PER_W
    rows = (rows0, rows1)
    osem = (osem0, osem1)

    cp_t = pltpu.async_copy(table_hbm, tbl, tsem)
    cp_x = pltpu.async_copy(x_hbm.at[pl.ds(base0, PER_W)], xbuf, isem)
    cp_t.wait()
    cp_x.wait()

    lane16 = lax.iota(jnp.int32, 16) * 16

    def fill(c, b):
        rbuf = rows[b]

        def grp(g, carry):
            idx16 = xbuf[pl.ds(c * CHUNK + g * 16, 16)]
            addr = idx16 * E
            dst = g * (16 * E) + lane16
            for e in range(E):
                vals = plsc.load_gather(tbl, [addr + e])
                plsc.store_scatter(rbuf, [dst + e], vals)
            return carry

        lax.fori_loop(0, CHUNK // 16, grp, 0)

    def out_start(c, b):
        return pltpu.async_copy(
            rows[b], out_hbm.at[pl.ds((base0 + c * CHUNK) * E, CHUNK * E)],
            osem[b])

    def out_wait(c, b):
        pltpu.make_async_copy(
            rows[b], out_hbm.at[pl.ds((base0 + c * CHUNK) * E, CHUNK * E)],
            osem[b]).wait()

    fill(0, 0)
    out_start(0, 0)
    fill(1, 1)
    out_start(1, 1)

    def lap(g, carry):
        for b in range(2):
            c = 2 * g + b
            out_wait(c - 2, b)
            fill(c, b)
            out_start(c, b)
        return carry

    lax.fori_loop(1, NCHUNK // 2, lap, 0)
    for b in range(2):
        out_wait(NCHUNK - 2 + b, b)


@functools.cache
def _sc_gather():
    return pl.kernel(
        _sc_gather_body,
        out_type=jax.ShapeDtypeStruct((NTOK * E,), jnp.float32),
        mesh=plsc.VectorSubcoreMesh(core_axis_name="c", subcore_axis_name="s"),
        scratch_types=[
            pltpu.VMEM((PER_W,), jnp.int32),
            pltpu.VMEM((VOCAB * E,), jnp.float32),
            pltpu.VMEM((CHUNK * E,), jnp.float32),
            pltpu.VMEM((CHUNK * E,), jnp.float32),
            pltpu.SemaphoreType.DMA,
            pltpu.SemaphoreType.DMA,
            pltpu.SemaphoreType.DMA,
            pltpu.SemaphoreType.DMA,
        ],
        compiler_params=pltpu.CompilerParams(use_tc_tiling_on_sc=False,
                                             needs_layout_passes=False),
    )


def _tc_body(z_ref, w_ref, b_ref, ws_ref, bs_ref, gw_ref, gb_ref,
             f1w_ref, f1b_ref, f2w_ref, f2b_ref, out_ref, m1_ref, m2_ref):
    i = pl.program_id(0)

    @pl.when(i == 0)
    def _init():
        m1_ref[...] = jnp.full((B, C), -jnp.inf, jnp.float32)
        m2_ref[...] = jnp.full((B, C), -jnp.inf, jnp.float32)

    a = z_ref[...].astype(jnp.bfloat16)
    c = jnp.dot(a, w_ref[...], preferred_element_type=jnp.float32)
    c = c + b_ref[...]
    u = c[:, :C] * jax.nn.sigmoid(c[:, C:2 * C])
    s = jnp.dot(u, ws_ref[...], preferred_element_type=jnp.float32)
    s = s + bs_ref[...]
    s = jnp.where(s >= 0.0, s, 0.01 * s)
    v = c[:, 2 * C:3 * C] * jax.nn.sigmoid(c[:, 3 * C:])

    m1_blk = jnp.max(s, axis=0, keepdims=True)
    m2_blk = jnp.max(v, axis=0, keepdims=True)
    row = lax.broadcasted_iota(jnp.int32, (B, 1), 0)
    sel = row == i
    m1_ref[...] = jnp.where(sel, jnp.maximum(m1_ref[...], m1_blk), m1_ref[...])
    m2_ref[...] = jnp.where(sel, jnp.maximum(m2_ref[...], m2_blk), m2_ref[...])

    @pl.when(i == pl.num_programs(0) - 1)
    def _head():
        gates = jax.nn.sigmoid(
            jnp.dot(m1_ref[...], gw_ref[...],
                    preferred_element_type=jnp.float32) + gb_ref[...])
        pooled = m2_ref[...] * gates
        f = jnp.dot(pooled, f1w_ref[...], preferred_element_type=jnp.float32)
        f = jnp.maximum(f + f1b_ref[...], 0.0)
        o = jnp.dot(f, f2w_ref[...], preferred_element_type=jnp.float32)
        out_ref[...] = o + f2b_ref[...]


def _full(shape):
    return pl.BlockSpec(shape, lambda i: (0, 0))


_tc_call = pl.pallas_call(
    _tc_body,
    grid=(GRID,),
    in_specs=[
        pl.BlockSpec((BM, KE), lambda i: (i, 0)),
        _full((KE, 4 * C)),
        _full((1, 4 * C)),
        _full((C, C)),
        _full((1, C)),
        _full((C, C)),
        _full((1, C)),
        _full((C, C)),
        _full((1, C)),
        _full((C, 128)),
        _full((1, 128)),
    ],
    out_specs=pl.BlockSpec((B, 128), lambda i: (0, 0)),
    out_shape=jax.ShapeDtypeStruct((B, 128), jnp.float32),
    scratch_shapes=[
        pltpu.VMEM((B, C), jnp.float32),
        pltpu.VMEM((B, C), jnp.float32),
    ],
)


def kernel(x, embed, ctx_conv_w, ctx_conv_b, ctx_share_w, ctx_share_b,
           gcg_conv_w, gcg_conv_b, gate_w, gate_b,
           fc1_w, fc1_b, fc2_w, fc2_b):
    zf = _sc_gather()(x.reshape(NTOK), embed.reshape(VOCAB * E))
    z = zf.reshape(NWIN, KE)

    wc = ctx_conv_w.transpose(2, 1, 0).reshape(KE, 2 * C)
    wg = gcg_conv_w.transpose(2, 1, 0).reshape(KE, 2 * C)
    w_all = jnp.concatenate([wc, wg], axis=1).astype(jnp.bfloat16)
    b_all = jnp.concatenate([ctx_conv_b, gcg_conv_b])[None, :]
    ws = ctx_share_w[:, :, 0].T
    bs = ctx_share_b[None, :]
    gw = gate_w.T
    gb = gate_b[None, :]
    f1w = fc1_w.T
    f1b = fc1_b[None, :]
    f2w = jnp.pad(fc2_w.T, ((0, 0), (0, 128 - fc2_w.shape[0])))
    f2b = jnp.pad(fc2_b, (0, 128 - fc2_b.shape[0]))[None, :]

    out = _tc_call(z, w_all, b_all, ws, bs, gw, gb, f1w, f1b, f2w, f2b)
    return out[:, :fc2_w.shape[0]]


# speedup vs baseline: 34.8760x; 1.3939x over previous
import functools

import jax
import jax.numpy as jnp
from jax import lax
from jax.experimental import pallas as pl
from jax.experimental.pallas import tpu as pltpu
from jax.experimental.pallas import tpu_sc as plsc

E = 16
C = 256
K = 512
S = 512
B = 8
T = 262144
VOCAB = 257
NTOK = B * T
NWIN = B * (T // S)
KE = K * E

NC = 2
NS = 16
NW = NC * NS
PER_W = NTOK // NW
CHUNK = 1024
NCHUNK = PER_W // CHUNK

BM = 512
GRID = NWIN // BM


def _sc_gather_body(x_hbm, table_hbm, out_hbm, xbuf, tbl, rows0, rows1,
                    isem, tsem, osem0, osem1):
    wid = lax.axis_index("s") * NC + lax.axis_index("c")
    base0 = wid * PER_W
    rows = (rows0, rows1)
    osem = (osem0, osem1)

    cp_t = pltpu.async_copy(table_hbm, tbl, tsem)
    cp_x = pltpu.async_copy(x_hbm.at[pl.ds(base0, PER_W)], xbuf, isem)
    cp_t.wait()
    cp_x.wait()

    def fill(c, b):
        rbuf = rows[b]

        def grp(g, carry):
            idx16 = xbuf[pl.ds(c * CHUNK + g * 16, 16)]
            base = g * (16 * E)
            for e in range(E):
                vals = plsc.load_gather(tbl, [idx16 + e * VOCAB])
                rbuf[pl.ds(base + e * 16, 16)] = vals
            return carry

        lax.fori_loop(0, CHUNK // 16, grp, 0)

    def out_start(c, b):
        return pltpu.async_copy(
            rows[b], out_hbm.at[pl.ds((base0 + c * CHUNK) * E, CHUNK * E)],
            osem[b])

    def out_wait(c, b):
        pltpu.make_async_copy(
            rows[b], out_hbm.at[pl.ds((base0 + c * CHUNK) * E, CHUNK * E)],
            osem[b]).wait()

    fill(0, 0)
    out_start(0, 0)
    fill(1, 1)
    out_start(1, 1)

    def lap(g, carry):
        for b in range(2):
            c = 2 * g + b
            out_wait(c - 2, b)
            fill(c, b)
            out_start(c, b)
        return carry

    lax.fori_loop(1, NCHUNK // 2, lap, 0)
    for b in range(2):
        out_wait(NCHUNK - 2 + b, b)


@functools.cache
def _sc_gather():
    return pl.kernel(
        _sc_gather_body,
        out_type=jax.ShapeDtypeStruct((NTOK * E,), jnp.float32),
        mesh=plsc.VectorSubcoreMesh(core_axis_name="c", subcore_axis_name="s"),
        scratch_types=[
            pltpu.VMEM((PER_W,), jnp.int32),
            pltpu.VMEM((VOCAB * E,), jnp.float32),
            pltpu.VMEM((CHUNK * E,), jnp.float32),
            pltpu.VMEM((CHUNK * E,), jnp.float32),
            pltpu.SemaphoreType.DMA,
            pltpu.SemaphoreType.DMA,
            pltpu.SemaphoreType.DMA,
            pltpu.SemaphoreType.DMA,
        ],
        compiler_params=pltpu.CompilerParams(use_tc_tiling_on_sc=False,
                                             needs_layout_passes=False),
    )


def _tc_body(z_ref, w_ref, b_ref, ws_ref, bs_ref, gw_ref, gb_ref,
             f1w_ref, f1b_ref, f2w_ref, f2b_ref, out_ref, m1_ref, m2_ref):
    i = pl.program_id(0)

    @pl.when(i == 0)
    def _init():
        m1_ref[...] = jnp.full((B, C), -jnp.inf, jnp.float32)
        m2_ref[...] = jnp.full((B, C), -jnp.inf, jnp.float32)

    a = z_ref[...].astype(jnp.bfloat16)
    c = jnp.dot(a, w_ref[...], preferred_element_type=jnp.float32)
    c = c + b_ref[...]
    u = c[:, :C] * jax.nn.sigmoid(c[:, C:2 * C])
    s = jnp.dot(u, ws_ref[...], preferred_element_type=jnp.float32)
    s = s + bs_ref[...]
    s = jnp.where(s >= 0.0, s, 0.01 * s)
    v = c[:, 2 * C:3 * C] * jax.nn.sigmoid(c[:, 3 * C:])

    m1_blk = jnp.max(s, axis=0, keepdims=True)
    m2_blk = jnp.max(v, axis=0, keepdims=True)
    row = lax.broadcasted_iota(jnp.int32, (B, 1), 0)
    sel = row == i
    m1_ref[...] = jnp.where(sel, jnp.maximum(m1_ref[...], m1_blk), m1_ref[...])
    m2_ref[...] = jnp.where(sel, jnp.maximum(m2_ref[...], m2_blk), m2_ref[...])

    @pl.when(i == pl.num_programs(0) - 1)
    def _head():
        gates = jax.nn.sigmoid(
            jnp.dot(m1_ref[...], gw_ref[...],
                    preferred_element_type=jnp.float32) + gb_ref[...])
        pooled = m2_ref[...] * gates
        f = jnp.dot(pooled, f1w_ref[...], preferred_element_type=jnp.float32)
        f = jnp.maximum(f + f1b_ref[...], 0.0)
        o = jnp.dot(f, f2w_ref[...], preferred_element_type=jnp.float32)
        out_ref[...] = o + f2b_ref[...]


def _full(shape):
    return pl.BlockSpec(shape, lambda i: (0, 0))


_tc_call = pl.pallas_call(
    _tc_body,
    grid=(GRID,),
    in_specs=[
        pl.BlockSpec((BM, KE), lambda i: (i, 0)),
        _full((KE, 4 * C)),
        _full((1, 4 * C)),
        _full((C, C)),
        _full((1, C)),
        _full((C, C)),
        _full((1, C)),
        _full((C, C)),
        _full((1, C)),
        _full((C, 128)),
        _full((1, 128)),
    ],
    out_specs=pl.BlockSpec((B, 128), lambda i: (0, 0)),
    out_shape=jax.ShapeDtypeStruct((B, 128), jnp.float32),
    scratch_shapes=[
        pltpu.VMEM((B, C), jnp.float32),
        pltpu.VMEM((B, C), jnp.float32),
    ],
)


def kernel(x, embed, ctx_conv_w, ctx_conv_b, ctx_share_w, ctx_share_b,
           gcg_conv_w, gcg_conv_b, gate_w, gate_b,
           fc1_w, fc1_b, fc2_w, fc2_b):
    zf = _sc_gather()(x.reshape(NTOK), embed.T.reshape(E * VOCAB))
    z = zf.reshape(NWIN, KE)

    wc = ctx_conv_w.reshape(2 * C, E, K // 16, 16).transpose(
        2, 1, 3, 0).reshape(KE, 2 * C)
    wg = gcg_conv_w.reshape(2 * C, E, K // 16, 16).transpose(
        2, 1, 3, 0).reshape(KE, 2 * C)
    w_all = jnp.concatenate([wc, wg], axis=1).astype(jnp.bfloat16)
    b_all = jnp.concatenate([ctx_conv_b, gcg_conv_b])[None, :]
    ws = ctx_share_w[:, :, 0].T
    bs = ctx_share_b[None, :]
    gw = gate_w.T
    gb = gate_b[None, :]
    f1w = fc1_w.T
    f1b = fc1_b[None, :]
    f2w = jnp.pad(fc2_w.T, ((0, 0), (0, 128 - fc2_w.shape[0])))
    f2b = jnp.pad(fc2_b, (0, 128 - fc2_b.shape[0]))[None, :]

    out = _tc_call(z, w_all, b_all, ws, bs, gw, gb, f1w, f1b, f2w, f2b)
    return out[:, :fc2_w.shape[0]]
